# baseline (speedup 1.0000x reference)
import functools

import jax
import jax.numpy as jnp
from jax import lax
from jax.experimental import pallas as pl
from jax.experimental.pallas import tpu as pltpu
from jax.experimental.pallas import tpu_sc as plsc

N = 50000
E = 800000
PERIODS = 12
HID = 32

NP = 50176
NPT = NP // 16
NW = 32
EA = E + N
ROWS_W = 216
ROWS = ROWS_W * NW
EP = ROWS * 128
KB = 24
NG = ROWS_W // KB
ROWS_T = ROWS // 16
NGD = ROWS_T // KB

_mesh = plsc.VectorSubcoreMesh(core_axis_name="c", subcore_axis_name="s")


@functools.partial(
    pl.kernel,
    out_type=jax.ShapeDtypeStruct((2, NP, 16), jnp.float32),
    mesh=_mesh,
    scratch_types=[
        pltpu.VMEM_SHARED((NP, 16), jnp.float32),
        pltpu.VMEM_SHARED((NP, 16), jnp.float32),
        pltpu.VMEM_SHARED((NP,), jnp.float32),
        pltpu.VMEM((KB, 128), jnp.int32),
        pltpu.VMEM((KB, 128), jnp.int32),
        pltpu.VMEM((KB, 128), jnp.float32),
        pltpu.VMEM((3, 128), jnp.float32),
        pltpu.VMEM((3, 128, 16), jnp.float32),
        pltpu.VMEM((2, 128, 16), jnp.float32),
        pltpu.VMEM((112, 16), jnp.float32),
        pltpu.VMEM((NPT,), jnp.float32),
        pltpu.SemaphoreType.DMA((3,)),
        pltpu.SemaphoreType.DMA((3,)),
        pltpu.SemaphoreType.DMA((2,)),
        pltpu.SemaphoreType.DMA,
    ],
    compiler_params=pltpu.CompilerParams(
        needs_layout_passes=False, use_tc_tiling_on_sc=False),
)
def _sc_kernel(src_hbm, dst_hbm, ew_hbm, x_hbm, out_hbm,
               s_sh, x_sh, deg_sh, srcbuf, dstbuf, ewbuf, dg, xg, xs,
               zbuf, dbuf, dsem, xsem, ssem, degsem):
    c = lax.axis_index("c")
    s = lax.axis_index("s")
    wid = s * 2 + c
    zero16 = jnp.zeros((16,), jnp.float32)

    def zd(i, _):
        dbuf[pl.ds(i * 16, 16)] = zero16
        return 0

    lax.fori_loop(0, NPT // 16, zd, 0)
    pltpu.sync_copy(dbuf, deg_sh.at[pl.ds(s * NPT, NPT)])

    def zb(i, _):
        zbuf[i, :] = zero16
        return 0

    lax.fori_loop(0, 112, zb, 0)
    for q in range(28):
        pltpu.sync_copy(zbuf, s_sh.at[pl.ds(s * NPT + q * 112, 112), :])
    for q in range(28):
        pltpu.sync_copy(x_hbm.at[pl.ds(s * NPT + q * 112, 112), :], zbuf)
        pltpu.sync_copy(zbuf, x_sh.at[pl.ds(s * NPT + q * 112, 112), :])
    plsc.subcore_barrier()

    def dgrp(g, _):
        base = s * ROWS_T + g * KB
        pltpu.sync_copy(dst_hbm.at[pl.ds(base, KB)], dstbuf)
        pltpu.sync_copy(ew_hbm.at[pl.ds(base, KB)], ewbuf)

        def fire(r, _):
            pltpu.async_copy(ewbuf.at[r], deg_sh.at[dstbuf.at[r]], degsem,
                             add=True)
            return 0

        lax.fori_loop(0, KB, fire, 0)

        def drain(r, _):
            pltpu.make_async_copy(ewbuf.at[r], deg_sh.at[dstbuf.at[r]],
                                  degsem).wait()
            return 0

        lax.fori_loop(0, KB, drain, 0)
        return 0

    lax.fori_loop(0, NGD, dgrp, 0)
    plsc.subcore_barrier()

    pltpu.sync_copy(deg_sh.at[pl.ds(s * NPT, NPT)], dbuf)

    def rsq(i, _):
        d = dbuf[pl.ds(i * 16, 16)]
        bits = plsc.bitcast(d, jnp.int32)
        y = plsc.bitcast(jnp.int32(0x5F3759DF) - lax.shift_right_logical(bits, 1),
                         jnp.float32)
        for _u in range(3):
            y = y * (1.5 - 0.5 * d * y * y)
        dbuf[pl.ds(i * 16, 16)] = y
        return 0

    lax.fori_loop(0, NPT // 16, rsq, 0)
    pltpu.sync_copy(dbuf, deg_sh.at[pl.ds(s * NPT, NPT)])
    plsc.subcore_barrier()

    def _issue_gathers(r, b):
        pltpu.async_copy(deg_sh.at[srcbuf.at[r]], dg.at[b], dsem.at[b])
        pltpu.async_copy(x_sh.at[srcbuf.at[r]], xg.at[b], xsem.at[b])

    def grp(g, _):
        base = wid * ROWS_W + g * KB
        pltpu.sync_copy(src_hbm.at[pl.ds(base, KB)], srcbuf)
        pltpu.sync_copy(dst_hbm.at[pl.ds(base, KB)], dstbuf)
        pltpu.sync_copy(ew_hbm.at[pl.ds(base, KB)], ewbuf)
        for r0 in range(3):
            _issue_gathers(r0, r0)

        def row(r, _):
            b = r % 3
            sb = r % 2
            pltpu.make_async_copy(deg_sh.at[srcbuf.at[r]], dg.at[b],
                                  dsem.at[b]).wait()
            pltpu.make_async_copy(x_sh.at[srcbuf.at[r]], xg.at[b],
                                  xsem.at[b]).wait()

            @pl.when(r >= 2)
            def _():
                pltpu.make_async_copy(xs.at[sb], s_sh.at[dstbuf.at[r]],
                                      ssem.at[sb]).wait()

            def sc(j, _):
                jb = j * 16
                w16 = dg[b, pl.ds(jb, 16)] * ewbuf[r, pl.ds(jb, 16)]
                for u in range(16):
                    xs[sb, jb + u, :] = xg[b, jb + u, :] * w16[u]
                return 0

            lax.fori_loop(0, 8, sc, 0)
            pltpu.async_copy(xs.at[sb], s_sh.at[dstbuf.at[r]], ssem.at[sb],
                             add=True)

            @pl.when(r + 3 < KB)
            def _():
                _issue_gathers(r + 3, b)

            return 0

        lax.fori_loop(0, KB, row, 0)
        pltpu.make_async_copy(xs.at[0], s_sh.at[dstbuf.at[0]], ssem.at[0]).wait()
        pltpu.make_async_copy(xs.at[1], s_sh.at[dstbuf.at[1]], ssem.at[1]).wait()
        return 0

    lax.fori_loop(0, NG, grp, 0)
    plsc.subcore_barrier()

    def wout(q, _):
        pltpu.sync_copy(s_sh.at[pl.ds(s * NPT + q * 112, 112), :], zbuf)

        def sc16(j, _):
            dv16 = dbuf[pl.ds(q * 112 + j * 16, 16)]
            for u in range(16):
                zbuf[j * 16 + u, :] = zbuf[j * 16 + u, :] * dv16[u]
            return 0

        lax.fori_loop(0, 7, sc16, 0)
        pltpu.sync_copy(zbuf, out_hbm.at[c, pl.ds(s * NPT + q * 112, 112), :])
        return 0

    lax.fori_loop(0, 28, wout, 0)


BB = 1792
NBLK = NP // BB


def _dense_body(p_ref, s_ref, out_ref):
    st = jnp.transpose(s_ref[0] + s_ref[1], (1, 0))
    az = p_ref[0, :].reshape(HID, 1)
    cz = p_ref[1, :].reshape(HID, 1)
    ah = p_ref[2, :].reshape(HID, 1)
    ch = p_ref[3, :].reshape(HID, 1)
    wl = p_ref[4, :].reshape(HID, 1)
    acc = jnp.zeros((HID, BB), jnp.float32)
    for p in range(PERIODS):
        sp = st[p:p + 1, :]
        z = jax.nn.sigmoid(az * sp + cz)
        ht = jnp.tanh(ah * sp + ch)
        acc = acc + p_ref[5, p] * ((1.0 - z) * ht)
    h = jnp.maximum(acc, 0.0)
    res = jnp.sum(h * wl, axis=0, keepdims=True) + p_ref[6, 0]
    out_ref[...] = jnp.transpose(res, (1, 0))


_dense_call = pl.pallas_call(
    _dense_body,
    grid=(NBLK,),
    in_specs=[
        pl.BlockSpec((8, HID), lambda i: (0, 0)),
        pl.BlockSpec((2, BB, 16), lambda i: (0, i, 0)),
    ],
    out_specs=pl.BlockSpec((BB, 1), lambda i: (i, 0)),
    out_shape=jax.ShapeDtypeStruct((NP, 1), jnp.float32),
)


def kernel(x, edge_index, edge_weight, Wcz, bcz, Wcr, bcr, Wch, bch,
           Wlz, blz, Wlr, blr, Wlh, blh, att, Wlin, blin):
    src = edge_index[0]
    dst = edge_index[1]
    pad = EP - EA
    loop = jnp.arange(N, dtype=jnp.int32)
    src_p = jnp.concatenate([src, loop, jnp.zeros((pad,), jnp.int32)])
    dst_p = jnp.concatenate([dst, loop,
                             (jnp.arange(pad, dtype=jnp.int32) * 41) % N])
    ew_p = jnp.concatenate([edge_weight, jnp.ones((N,), jnp.float32),
                            jnp.zeros((pad,), jnp.float32)])
    src2 = src_p.reshape(ROWS, 128)
    dst2 = dst_p.reshape(ROWS, 128)
    ew2 = ew_p.reshape(ROWS, 128)
    x_pad = jnp.pad(x, ((0, NP - N), (0, 16 - PERIODS)))

    s2 = _sc_kernel(src2, dst2, ew2, x_pad)

    hp = jax.lax.Precision.HIGHEST
    wlz_t = Wlz[:HID]
    wlh_t = Wlh[:HID]
    az = jnp.matmul(Wcz, wlz_t, precision=hp)[0]
    cz = jnp.matmul(bcz.reshape(1, HID), wlz_t, precision=hp)[0] + blz
    ah = jnp.matmul(Wch, wlh_t, precision=hp)[0]
    ch = jnp.matmul(bch.reshape(1, HID), wlh_t, precision=hp)[0] + blh
    probs = jax.nn.softmax(att)
    params = jnp.stack([
        az, cz, ah, ch, Wlin[:, 0],
        jnp.pad(probs, (0, HID - PERIODS)),
        jnp.full((HID,), blin[0], jnp.float32),
        jnp.zeros((HID,), jnp.float32),
    ]).astype(jnp.float32)

    out = _dense_call(params, s2)
    return out[:N]
